# Initial kernel scaffold; baseline (speedup 1.0000x reference)
import jax
import jax.numpy as jnp
from jax.experimental import pallas as pl


def kernel(input_data, obs_idx, next_idx, gap, W_in, b_in, W_time, W_q, W_k, W_v, W_o, W_ff1, b_ff1, W_ff2, b_ff2, W_out, b_out):
    raise NotImplementedError("write your pallas kernel here")


# speedup vs baseline: 1.6150x; 1.6150x over previous
import functools

import numpy as np
import jax
import jax.numpy as jnp
from jax import lax
from jax.experimental import pallas as pl
from jax.experimental.pallas import tpu as pltpu
from jax.experimental.pallas import tpu_sc as plsc

_B = 16
_T = 512
_P = 10
_F = 6
_PF = _P * _F
_DM = 64
_TDIM = 64
_NH = 4
_DH = 16
_DI = 128
_NOBS = 256
_NNEXT = 256

_NC = 2
_NS = 16
_NW = _NC * _NS
_ROWS = _B * _NOBS
_RPW = _ROWS // _NW


_SLAB = _T * _PF
_CHUNK = _RPW * _PF
_NGRP = _CHUNK // 16


def _sc_gather(x2d, obs_idx, next_idx):
    mesh = plsc.VectorSubcoreMesh(core_axis_name="c", subcore_axis_name="s")

    @functools.partial(
        pl.kernel,
        mesh=mesh,
        out_type=(
            jax.ShapeDtypeStruct((_ROWS * _PF,), jnp.float32),
            jax.ShapeDtypeStruct((_ROWS * _PF,), jnp.float32),
        ),
        scratch_types=[
            pltpu.VMEM((_SLAB,), jnp.float32),
            pltpu.VMEM((_RPW,), jnp.int32),
            pltpu.VMEM((_RPW,), jnp.int32),
            pltpu.VMEM((_CHUNK,), jnp.float32),
            pltpu.SemaphoreType.DMA,
        ],
        compiler_params=pltpu.CompilerParams(
            use_tc_tiling_on_sc=False, needs_layout_passes=False),
    )
    def k(x_hbm, obs_hbm, nxt_hbm, out_obs, out_gt, slab_v, ti_v, tn_v, buf_v, sem):
        w = lax.axis_index("s") * _NC + lax.axis_index("c")
        b = w // 2
        n0 = pl.multiple_of((w % 2) * _RPW, _RPW)
        base = pl.multiple_of(w * _CHUNK, _CHUNK)
        pltpu.sync_copy(x_hbm.at[b], slab_v)
        pltpu.sync_copy(obs_hbm.at[pl.ds(n0, _RPW)], ti_v)
        pltpu.sync_copy(nxt_hbm.at[pl.ds(n0, _RPW)], tn_v)
        lane = lax.iota(jnp.int32, 16)
        for ti, out_hbm in ((ti_v, out_obs), (tn_v, out_gt)):
            @plsc.parallel_loop(0, _NGRP, step=1, unroll=8)
            def _g(g):
                o = g * 16 + lane
                n = lax.shift_right_logical(o * 34953, 21)
                f = o - n * _PF
                t16 = plsc.load_gather(ti, [n])
                v = plsc.load_gather(slab_v, [t16 * _PF + f])
                buf_v[pl.ds(pl.multiple_of(g * 16, 16), 16)] = v
            pltpu.sync_copy(buf_v, out_hbm.at[pl.ds(base, _CHUNK)])

    return k(x2d.reshape(_B, _SLAB), obs_idx, next_idx)


def _tc_body(obs_r, gt_r, opos_r, npos_r, ig_r,
             w_in_r, w_time_r, w_q_r, w_k_r, w_v_r, w_o_r,
             w_ff1_r, w_ff2_r, w_out_r,
             b_in_r, b_ff1_r, b_ff2_r, b_out_r,
             out_r, loss_r):
    b = pl.program_id(0)
    ig = ig_r[0, 0]

    i2 = lax.broadcasted_iota(jnp.int32, (_NOBS, _TDIM // 2), 1).astype(jnp.float32)
    freq = jnp.exp(i2 * jnp.float32(-2.0 * np.log(10000.0) / _TDIM))

    def tenc(pos_col):
        ang = (pos_col * ig) * freq
        return jnp.concatenate([jnp.sin(ang), jnp.cos(ang)], axis=1)

    w_time = w_time_r[...]
    t_obs = tenc(opos_r[...])
    t_next = tenc(npos_r[...])
    c_obs = jnp.dot(t_obs, w_time, preferred_element_type=jnp.float32) + b_in_r[...]
    q_base = jnp.dot(t_next, w_time, preferred_element_type=jnp.float32)
    q = jnp.dot(q_base, w_q_r[...], preferred_element_type=jnp.float32)

    w_in = w_in_r[...]
    w_k = w_k_r[...]
    w_v = w_v_r[...]
    w_o = w_o_r[...]
    w_ff1 = w_ff1_r[...]
    w_ff2 = w_ff2_r[...]
    w_out = w_out_r[...]
    b_ff1 = b_ff1_r[...]
    b_ff2 = b_ff2_r[...]
    b_out = b_out_r[...]

    obs_all = obs_r[0]
    outs = []
    for p in range(_P):
        op = obs_all[:, _F * p:_F * (p + 1)]
        h0 = jnp.dot(op, w_in, preferred_element_type=jnp.float32) + c_obs
        kk = jnp.dot(h0, w_k, preferred_element_type=jnp.float32)
        vv = jnp.dot(h0, w_v, preferred_element_type=jnp.float32)
        ctxs = []
        for hh in range(_NH):
            qh = q[:, _DH * hh:_DH * (hh + 1)]
            kh = kk[:, _DH * hh:_DH * (hh + 1)]
            vh = vv[:, _DH * hh:_DH * (hh + 1)]
            lg = lax.dot_general(qh, kh, (((1,), (1,)), ((), ())),
                                 preferred_element_type=jnp.float32)
            lg = lg * jnp.float32(1.0 / np.sqrt(_DH))
            m = jnp.max(lg, axis=1, keepdims=True)
            e = jnp.exp(lg - m)
            s = jnp.sum(e, axis=1, keepdims=True)
            ctxs.append(jnp.dot(e, vh, preferred_element_type=jnp.float32) / s)
        ctx = jnp.concatenate(ctxs, axis=1)
        h1 = q_base + jnp.dot(ctx, w_o, preferred_element_type=jnp.float32)
        ff = jnp.maximum(jnp.dot(h1, w_ff1, preferred_element_type=jnp.float32) + b_ff1, 0.0)
        h2 = h1 + jnp.dot(ff, w_ff2, preferred_element_type=jnp.float32) + b_ff2
        outs.append(jnp.dot(h2, w_out, preferred_element_type=jnp.float32) + b_out)
    o = jnp.concatenate(outs, axis=1)
    out_r[0] = o

    part = jnp.sum(jnp.abs(o - gt_r[0]))

    @pl.when(b == 0)
    def _init():
        loss_r[0, 0] = 0.0

    loss_r[0, 0] += part

    @pl.when(b == _B - 1)
    def _fin():
        loss_r[0, 0] = loss_r[0, 0] * jnp.float32(1.0 / (_B * _P * _NNEXT * _F))


def _tc_main(obs_rows, gt_rows, opos, npos, inv_gap,
             W_in, W_time, W_q, W_k, W_v, W_o, W_ff1, W_ff2, W_out,
             b_in, b_ff1, b_ff2, b_out):
    w2 = lambda b: (0, 0)
    in_specs = [
        pl.BlockSpec((1, _NOBS, _PF), lambda b: (b, 0, 0)),
        pl.BlockSpec((1, _NNEXT, _PF), lambda b: (b, 0, 0)),
        pl.BlockSpec((_NOBS, 1), w2),
        pl.BlockSpec((_NNEXT, 1), w2),
        pl.BlockSpec((1, 1), w2),
        pl.BlockSpec((_F, _DM), w2),
        pl.BlockSpec((_TDIM, _DM), w2),
        pl.BlockSpec((_DM, _DM), w2),
        pl.BlockSpec((_DM, _DM), w2),
        pl.BlockSpec((_DM, _DM), w2),
        pl.BlockSpec((_DM, _DM), w2),
        pl.BlockSpec((_DM, _DI), w2),
        pl.BlockSpec((_DI, _DM), w2),
        pl.BlockSpec((_DM, _F), w2),
        pl.BlockSpec((1, _DM), w2),
        pl.BlockSpec((1, _DI), w2),
        pl.BlockSpec((1, _DM), w2),
        pl.BlockSpec((1, _F), w2),
    ]
    out_specs = [
        pl.BlockSpec((1, _NNEXT, _PF), lambda b: (b, 0, 0)),
        pl.BlockSpec((1, 1), w2, memory_space=pltpu.SMEM),
    ]
    out_shape = [
        jax.ShapeDtypeStruct((_B, _NNEXT, _PF), jnp.float32),
        jax.ShapeDtypeStruct((1, 1), jnp.float32),
    ]
    return pl.pallas_call(
        _tc_body,
        grid=(_B,),
        in_specs=in_specs,
        out_specs=out_specs,
        out_shape=out_shape,
        compiler_params=pltpu.CompilerParams(
            dimension_semantics=("arbitrary",),
        ),
    )(obs_rows, gt_rows, opos, npos, inv_gap,
      W_in, W_time, W_q, W_k, W_v, W_o, W_ff1, W_ff2, W_out,
      b_in, b_ff1, b_ff2, b_out)


def kernel(input_data, obs_idx, next_idx, gap,
           W_in, b_in, W_time, W_q, W_k, W_v, W_o,
           W_ff1, b_ff1, W_ff2, b_ff2, W_out, b_out):
    x2d = input_data.reshape(_B * _T, _PF)
    r_obs, r_gt = _sc_gather(x2d, obs_idx, next_idx)
    obs_rows = r_obs.reshape(_B, _NOBS, _PF)
    gt_rows = r_gt.reshape(_B, _NNEXT, _PF)

    opos = obs_idx.astype(jnp.float32).reshape(_NOBS, 1)
    npos = next_idx.astype(jnp.float32).reshape(_NNEXT, 1)
    inv_gap = (1.0 / jnp.maximum(jnp.asarray(gap, jnp.float32), 1.0)).reshape(1, 1)

    out, loss = _tc_main(
        obs_rows, gt_rows, opos, npos, inv_gap,
        W_in, W_time, W_q, W_k, W_v, W_o, W_ff1, W_ff2, W_out,
        b_in.reshape(1, _DM), b_ff1.reshape(1, _DI),
        b_ff2.reshape(1, _DM), b_out.reshape(1, _F),
    )
    return (out, loss.reshape(())[()])


# speedup vs baseline: 2.6860x; 1.6631x over previous
import functools

import numpy as np
import jax
import jax.numpy as jnp
from jax import lax
from jax.experimental import pallas as pl
from jax.experimental.pallas import tpu as pltpu
from jax.experimental.pallas import tpu_sc as plsc

_B = 16
_T = 512
_P = 10
_F = 6
_PF = _P * _F
_DM = 64
_TDIM = 64
_NH = 4
_DH = 16
_DI = 128
_NOBS = 256
_NNEXT = 256

_NC = 2
_NS = 16
_NW = _NC * _NS
_ROWS = _B * _NOBS
_RPW = _ROWS // _NW


_SLAB = _T * _PF
_CHUNK = _RPW * _PF
_NGRP = _CHUNK // 16


def _sc_gather(x2d, obs_idx, next_idx):
    mesh = plsc.VectorSubcoreMesh(core_axis_name="c", subcore_axis_name="s")

    @functools.partial(
        pl.kernel,
        mesh=mesh,
        out_type=(
            jax.ShapeDtypeStruct((_ROWS * _PF,), jnp.float32),
            jax.ShapeDtypeStruct((_ROWS * _PF,), jnp.float32),
        ),
        scratch_types=[
            pltpu.VMEM((_SLAB,), jnp.float32),
            pltpu.VMEM((_RPW,), jnp.int32),
            pltpu.VMEM((_RPW,), jnp.int32),
            pltpu.VMEM((_CHUNK,), jnp.float32),
            pltpu.SemaphoreType.DMA,
        ],
        compiler_params=pltpu.CompilerParams(
            use_tc_tiling_on_sc=False, needs_layout_passes=False),
    )
    def k(x_hbm, obs_hbm, nxt_hbm, out_obs, out_gt, slab_v, ti_v, tn_v, buf_v, sem):
        w = lax.axis_index("s") * _NC + lax.axis_index("c")
        b = w // 2
        n0 = pl.multiple_of((w % 2) * _RPW, _RPW)
        base = pl.multiple_of(w * _CHUNK, _CHUNK)
        pltpu.sync_copy(x_hbm.at[b], slab_v)
        pltpu.sync_copy(obs_hbm.at[pl.ds(n0, _RPW)], ti_v)
        pltpu.sync_copy(nxt_hbm.at[pl.ds(n0, _RPW)], tn_v)
        lane = lax.iota(jnp.int32, 16)
        for ti, out_hbm in ((ti_v, out_obs), (tn_v, out_gt)):
            @plsc.parallel_loop(0, _NGRP, step=1, unroll=8)
            def _g(g):
                o = g * 16 + lane
                n = lax.shift_right_logical(o * 34953, 21)
                f = o - n * _PF
                t16 = plsc.load_gather(ti, [n])
                v = plsc.load_gather(slab_v, [t16 * _PF + f])
                buf_v[pl.ds(pl.multiple_of(g * 16, 16), 16)] = v
            pltpu.sync_copy(buf_v, out_hbm.at[pl.ds(base, _CHUNK)])

    return k(x2d.reshape(_B, _SLAB), obs_idx, next_idx)


def _tc_body(obs_r, gt_r, opos_r, npos_r, ig_r,
             w_in_r, w_time_r, w_q_r, w_k_r, w_v_r, w_o_r,
             w_ff1_r, w_ff2_r, w_out_r,
             b_in_r, b_ff1_r, b_ff2_r, b_out_r,
             out_r, loss_r,
             c_obs_s, qb_s, q_s, wkv_s):
    b = pl.program_id(0)

    @pl.when(b == 0)
    def _pre():
        ig = ig_r[0, 0]
        i2 = lax.broadcasted_iota(jnp.int32, (_NOBS, _TDIM // 2), 1).astype(jnp.float32)
        freq = jnp.exp(i2 * jnp.float32(-2.0 * np.log(10000.0) / _TDIM))

        def tenc(pos_col):
            ang = (pos_col * ig) * freq
            return jnp.concatenate([jnp.sin(ang), jnp.cos(ang)], axis=1)

        w_time = w_time_r[...]
        t_obs = tenc(opos_r[...])
        t_next = tenc(npos_r[...])
        c_obs_s[...] = jnp.dot(t_obs, w_time, preferred_element_type=jnp.float32) + b_in_r[...]
        qb = jnp.dot(t_next, w_time, preferred_element_type=jnp.float32)
        qb_s[...] = qb
        q_s[...] = jnp.dot(qb, w_q_r[...], preferred_element_type=jnp.float32)
        wkv_s[...] = jnp.concatenate([w_k_r[...], w_v_r[...]], axis=1)

    c_obs = c_obs_s[...]
    q_base = qb_s[...]
    q = q_s[...]
    wkv = wkv_s[...]

    w_in = w_in_r[...]
    w_o = w_o_r[...]
    w_ff1 = w_ff1_r[...]
    w_ff2 = w_ff2_r[...]
    w_out = w_out_r[...]
    b_ff1 = b_ff1_r[...]
    b_ff2 = b_ff2_r[...]
    b_out = b_out_r[...]
    ones_col = jnp.ones((_NOBS, 1), jnp.float32)

    obs_all = obs_r[0]
    outs = []
    for p in range(_P):
        op = obs_all[:, _F * p:_F * (p + 1)]
        h0 = jnp.dot(op, w_in, preferred_element_type=jnp.float32) + c_obs
        kv = jnp.dot(h0, wkv, preferred_element_type=jnp.float32)
        ctxs = []
        for hh in range(_NH):
            qh = q[:, _DH * hh:_DH * (hh + 1)]
            kh = kv[:, _DH * hh:_DH * (hh + 1)]
            vh = kv[:, _DM + _DH * hh:_DM + _DH * (hh + 1)]
            lg = lax.dot_general(qh, kh, (((1,), (1,)), ((), ())),
                                 preferred_element_type=jnp.float32)
            e = jnp.exp(lg * jnp.float32(1.0 / np.sqrt(_DH)))
            vh1 = jnp.concatenate([vh, ones_col], axis=1)
            cs = jnp.dot(e, vh1, preferred_element_type=jnp.float32)
            ctxs.append(cs[:, :_DH] / cs[:, _DH:_DH + 1])
        ctx = jnp.concatenate(ctxs, axis=1)
        h1 = q_base + jnp.dot(ctx, w_o, preferred_element_type=jnp.float32)
        ff = jnp.maximum(jnp.dot(h1, w_ff1, preferred_element_type=jnp.float32) + b_ff1, 0.0)
        h2 = h1 + jnp.dot(ff, w_ff2, preferred_element_type=jnp.float32) + b_ff2
        outs.append(jnp.dot(h2, w_out, preferred_element_type=jnp.float32) + b_out)
    o = jnp.concatenate(outs, axis=1)
    out_r[0] = o

    part = jnp.sum(jnp.abs(o - gt_r[0]))

    @pl.when(b == 0)
    def _init():
        loss_r[0, 0] = 0.0

    loss_r[0, 0] += part

    @pl.when(b == _B - 1)
    def _fin():
        loss_r[0, 0] = loss_r[0, 0] * jnp.float32(1.0 / (_B * _P * _NNEXT * _F))


def _tc_main(obs_rows, gt_rows, opos, npos, inv_gap,
             W_in, W_time, W_q, W_k, W_v, W_o, W_ff1, W_ff2, W_out,
             b_in, b_ff1, b_ff2, b_out):
    w2 = lambda b: (0, 0)
    in_specs = [
        pl.BlockSpec((1, _NOBS, _PF), lambda b: (b, 0, 0)),
        pl.BlockSpec((1, _NNEXT, _PF), lambda b: (b, 0, 0)),
        pl.BlockSpec((_NOBS, 1), w2),
        pl.BlockSpec((_NNEXT, 1), w2),
        pl.BlockSpec((1, 1), w2),
        pl.BlockSpec((_F, _DM), w2),
        pl.BlockSpec((_TDIM, _DM), w2),
        pl.BlockSpec((_DM, _DM), w2),
        pl.BlockSpec((_DM, _DM), w2),
        pl.BlockSpec((_DM, _DM), w2),
        pl.BlockSpec((_DM, _DM), w2),
        pl.BlockSpec((_DM, _DI), w2),
        pl.BlockSpec((_DI, _DM), w2),
        pl.BlockSpec((_DM, _F), w2),
        pl.BlockSpec((1, _DM), w2),
        pl.BlockSpec((1, _DI), w2),
        pl.BlockSpec((1, _DM), w2),
        pl.BlockSpec((1, _F), w2),
    ]
    out_specs = [
        pl.BlockSpec((1, _NNEXT, _PF), lambda b: (b, 0, 0)),
        pl.BlockSpec((1, 1), w2, memory_space=pltpu.SMEM),
    ]
    out_shape = [
        jax.ShapeDtypeStruct((_B, _NNEXT, _PF), jnp.float32),
        jax.ShapeDtypeStruct((1, 1), jnp.float32),
    ]
    return pl.pallas_call(
        _tc_body,
        grid=(_B,),
        in_specs=in_specs,
        out_specs=out_specs,
        out_shape=out_shape,
        scratch_shapes=[
            pltpu.VMEM((_NOBS, _DM), jnp.float32),
            pltpu.VMEM((_NNEXT, _DM), jnp.float32),
            pltpu.VMEM((_NNEXT, _DM), jnp.float32),
            pltpu.VMEM((_DM, 2 * _DM), jnp.float32),
        ],
        compiler_params=pltpu.CompilerParams(
            dimension_semantics=("arbitrary",),
        ),
    )(obs_rows, gt_rows, opos, npos, inv_gap,
      W_in, W_time, W_q, W_k, W_v, W_o, W_ff1, W_ff2, W_out,
      b_in, b_ff1, b_ff2, b_out)


def kernel(input_data, obs_idx, next_idx, gap,
           W_in, b_in, W_time, W_q, W_k, W_v, W_o,
           W_ff1, b_ff1, W_ff2, b_ff2, W_out, b_out):
    x2d = input_data.reshape(_B * _T, _PF)
    r_obs, r_gt = _sc_gather(x2d, obs_idx, next_idx)
    obs_rows = r_obs.reshape(_B, _NOBS, _PF)
    gt_rows = r_gt.reshape(_B, _NNEXT, _PF)

    opos = obs_idx.astype(jnp.float32).reshape(_NOBS, 1)
    npos = next_idx.astype(jnp.float32).reshape(_NNEXT, 1)
    inv_gap = (1.0 / jnp.maximum(jnp.asarray(gap, jnp.float32), 1.0)).reshape(1, 1)

    out, loss = _tc_main(
        obs_rows, gt_rows, opos, npos, inv_gap,
        W_in, W_time, W_q, W_k, W_v, W_o, W_ff1, W_ff2, W_out,
        b_in.reshape(1, _DM), b_ff1.reshape(1, _DI),
        b_ff2.reshape(1, _DM), b_out.reshape(1, _F),
    )
    return (out, loss.reshape(())[()])


# speedup vs baseline: 2.7721x; 1.0321x over previous
import functools

import numpy as np
import jax
import jax.numpy as jnp
from jax import lax
from jax.experimental import pallas as pl
from jax.experimental.pallas import tpu as pltpu
from jax.experimental.pallas import tpu_sc as plsc

_B = 16
_T = 512
_P = 10
_F = 6
_PF = _P * _F
_DM = 64
_TDIM = 64
_NH = 4
_DH = 16
_DI = 128
_NOBS = 256
_NNEXT = 256

_NC = 2
_NS = 16
_NW = _NC * _NS
_ROWS = _B * _NOBS
_RPW = _ROWS // _NW


_SLAB = _T * _PF
_CHUNK = _RPW * _PF
_NGRP = _CHUNK // 16


def _sc_gather(x2d, obs_idx, next_idx):
    mesh = plsc.VectorSubcoreMesh(core_axis_name="c", subcore_axis_name="s")

    @functools.partial(
        pl.kernel,
        mesh=mesh,
        out_type=(
            jax.ShapeDtypeStruct((_ROWS * _PF,), jnp.float32),
            jax.ShapeDtypeStruct((_ROWS * _PF,), jnp.float32),
        ),
        scratch_types=[
            pltpu.VMEM((_SLAB,), jnp.float32),
            pltpu.VMEM((_RPW,), jnp.int32),
            pltpu.VMEM((_RPW,), jnp.int32),
            pltpu.VMEM((_CHUNK,), jnp.float32),
            pltpu.SemaphoreType.DMA,
        ],
        compiler_params=pltpu.CompilerParams(
            use_tc_tiling_on_sc=False, needs_layout_passes=False),
    )
    def k(x_hbm, obs_hbm, nxt_hbm, out_obs, out_gt, slab_v, ti_v, tn_v, buf_v, sem):
        w = lax.axis_index("s") * _NC + lax.axis_index("c")
        b = w // 2
        n0 = pl.multiple_of((w % 2) * _RPW, _RPW)
        base = pl.multiple_of(w * _CHUNK, _CHUNK)
        pltpu.sync_copy(x_hbm.at[b], slab_v)
        pltpu.sync_copy(obs_hbm.at[pl.ds(n0, _RPW)], ti_v)
        pltpu.sync_copy(nxt_hbm.at[pl.ds(n0, _RPW)], tn_v)
        lane = lax.iota(jnp.int32, 16)
        for ti, out_hbm in ((ti_v, out_obs), (tn_v, out_gt)):
            @plsc.parallel_loop(0, _NGRP, step=1, unroll=8)
            def _g(g):
                o = g * 16 + lane
                n = lax.shift_right_logical(o * 34953, 21)
                f = o - n * _PF
                t16 = plsc.load_gather(ti, [n])
                v = plsc.load_gather(slab_v, [t16 * _PF + f])
                buf_v[pl.ds(pl.multiple_of(g * 16, 16), 16)] = v
            pltpu.sync_copy(buf_v, out_hbm.at[pl.ds(base, _CHUNK)])

    return k(x2d.reshape(_B, _SLAB), obs_idx, next_idx)


def _tc_body(obs_r, gt_r, opos_r, npos_r, ig_r,
             w_in_r, w_time_r, w_q_r, w_k_r, w_v_r, w_o_r,
             w_ff1_r, w_ff2_r, w_out_r,
             b_in_r, b_ff1_r, b_ff2_r, b_out_r,
             out_r, loss_r,
             c_obs_s, qb_s, q_s, wkv_s,
             w_in_s, w_o_s, w_ff1_s, w_ff2_s, w_out_s):
    b = pl.program_id(0)
    bf = jnp.bfloat16

    @pl.when(b == 0)
    def _pre():
        ig = ig_r[0, 0]
        i2 = lax.broadcasted_iota(jnp.int32, (_NOBS, _TDIM // 2), 1).astype(jnp.float32)
        freq = jnp.exp(i2 * jnp.float32(-2.0 * np.log(10000.0) / _TDIM))

        def tenc(pos_col):
            ang = (pos_col * ig) * freq
            return jnp.concatenate([jnp.sin(ang), jnp.cos(ang)], axis=1)

        w_time = w_time_r[...]
        t_obs = tenc(opos_r[...])
        t_next = tenc(npos_r[...])
        c_obs_s[...] = jnp.dot(t_obs, w_time, preferred_element_type=jnp.float32) + b_in_r[...]
        qb = jnp.dot(t_next, w_time, preferred_element_type=jnp.float32)
        qb_s[...] = qb
        qsc = jnp.float32(np.log2(np.e) / np.sqrt(_DH))
        q_s[...] = (jnp.dot(qb, w_q_r[...], preferred_element_type=jnp.float32) * qsc).astype(bf)
        wkv_s[...] = jnp.concatenate([w_k_r[...], w_v_r[...]], axis=1).astype(bf)
        w_in_s[...] = w_in_r[...].astype(bf)
        w_o_s[...] = w_o_r[...].astype(bf)
        w_ff1_s[...] = w_ff1_r[...].astype(bf)
        w_ff2_s[...] = w_ff2_r[...].astype(bf)
        w_out_s[...] = w_out_r[...].astype(bf)

    c_obs = c_obs_s[...]
    q_base = qb_s[...]
    q = q_s[...]
    wkv = wkv_s[...]

    w_in = w_in_s[...]
    w_o = w_o_s[...]
    w_ff1 = w_ff1_s[...]
    w_ff2 = w_ff2_s[...]
    w_out = w_out_s[...]
    b_ff1 = b_ff1_r[...]
    b_ff2 = b_ff2_r[...]
    b_out = b_out_r[...]
    ones_col = jnp.ones((_NOBS, 1), bf)

    def fdot(a, b2):
        return jnp.dot(a, b2, preferred_element_type=jnp.float32)

    obs_all = obs_r[0]
    outs = []
    for p in range(_P):
        op = obs_all[:, _F * p:_F * (p + 1)].astype(bf)
        h0 = fdot(op, w_in) + c_obs
        kvb = fdot(h0.astype(bf), wkv).astype(bf)
        v1 = jnp.concatenate([kvb[:, _DM:], ones_col], axis=1)
        ctxs = []
        for hh in range(_NH):
            qh = q[:, _DH * hh:_DH * (hh + 1)]
            kh = kvb[:, _DH * hh:_DH * (hh + 1)]
            lg = lax.dot_general(qh, kh, (((1,), (1,)), ((), ())),
                                 preferred_element_type=jnp.float32)
            e = jnp.exp2(lg.astype(bf))
            cs = fdot(e, v1)
            s = cs[:, _DM:_DM + 1]
            ctxs.append(cs[:, _DH * hh:_DH * (hh + 1)] * (1.0 / s))
        ctx = jnp.concatenate(ctxs, axis=1)
        h1 = q_base + fdot(ctx.astype(bf), w_o)
        ff = jnp.maximum(fdot(h1.astype(bf), w_ff1) + b_ff1, 0.0)
        h2 = h1 + fdot(ff.astype(bf), w_ff2) + b_ff2
        outs.append(fdot(h2.astype(bf), w_out) + b_out)
    o = jnp.concatenate(outs, axis=1)
    out_r[0] = o

    part = jnp.sum(jnp.abs(o - gt_r[0]))

    @pl.when(b == 0)
    def _init():
        loss_r[0, 0] = 0.0

    loss_r[0, 0] += part

    @pl.when(b == _B - 1)
    def _fin():
        loss_r[0, 0] = loss_r[0, 0] * jnp.float32(1.0 / (_B * _P * _NNEXT * _F))


def _tc_main(obs_rows, gt_rows, opos, npos, inv_gap,
             W_in, W_time, W_q, W_k, W_v, W_o, W_ff1, W_ff2, W_out,
             b_in, b_ff1, b_ff2, b_out):
    w2 = lambda b: (0, 0)
    in_specs = [
        pl.BlockSpec((1, _NOBS, _PF), lambda b: (b, 0, 0)),
        pl.BlockSpec((1, _NNEXT, _PF), lambda b: (b, 0, 0)),
        pl.BlockSpec((_NOBS, 1), w2),
        pl.BlockSpec((_NNEXT, 1), w2),
        pl.BlockSpec((1, 1), w2),
        pl.BlockSpec((_F, _DM), w2),
        pl.BlockSpec((_TDIM, _DM), w2),
        pl.BlockSpec((_DM, _DM), w2),
        pl.BlockSpec((_DM, _DM), w2),
        pl.BlockSpec((_DM, _DM), w2),
        pl.BlockSpec((_DM, _DM), w2),
        pl.BlockSpec((_DM, _DI), w2),
        pl.BlockSpec((_DI, _DM), w2),
        pl.BlockSpec((_DM, _F), w2),
        pl.BlockSpec((1, _DM), w2),
        pl.BlockSpec((1, _DI), w2),
        pl.BlockSpec((1, _DM), w2),
        pl.BlockSpec((1, _F), w2),
    ]
    out_specs = [
        pl.BlockSpec((1, _NNEXT, _PF), lambda b: (b, 0, 0)),
        pl.BlockSpec((1, 1), w2, memory_space=pltpu.SMEM),
    ]
    out_shape = [
        jax.ShapeDtypeStruct((_B, _NNEXT, _PF), jnp.float32),
        jax.ShapeDtypeStruct((1, 1), jnp.float32),
    ]
    return pl.pallas_call(
        _tc_body,
        grid=(_B,),
        in_specs=in_specs,
        out_specs=out_specs,
        out_shape=out_shape,
        scratch_shapes=[
            pltpu.VMEM((_NOBS, _DM), jnp.float32),
            pltpu.VMEM((_NNEXT, _DM), jnp.float32),
            pltpu.VMEM((_NNEXT, _DM), jnp.bfloat16),
            pltpu.VMEM((_DM, 2 * _DM), jnp.bfloat16),
            pltpu.VMEM((_F, _DM), jnp.bfloat16),
            pltpu.VMEM((_DM, _DM), jnp.bfloat16),
            pltpu.VMEM((_DM, _DI), jnp.bfloat16),
            pltpu.VMEM((_DI, _DM), jnp.bfloat16),
            pltpu.VMEM((_DM, _F), jnp.bfloat16),
        ],
        compiler_params=pltpu.CompilerParams(
            dimension_semantics=("arbitrary",),
        ),
    )(obs_rows, gt_rows, opos, npos, inv_gap,
      W_in, W_time, W_q, W_k, W_v, W_o, W_ff1, W_ff2, W_out,
      b_in, b_ff1, b_ff2, b_out)


def kernel(input_data, obs_idx, next_idx, gap,
           W_in, b_in, W_time, W_q, W_k, W_v, W_o,
           W_ff1, b_ff1, W_ff2, b_ff2, W_out, b_out):
    x2d = input_data.reshape(_B * _T, _PF)
    r_obs, r_gt = _sc_gather(x2d, obs_idx, next_idx)
    obs_rows = r_obs.reshape(_B, _NOBS, _PF)
    gt_rows = r_gt.reshape(_B, _NNEXT, _PF)

    opos = obs_idx.astype(jnp.float32).reshape(_NOBS, 1)
    npos = next_idx.astype(jnp.float32).reshape(_NNEXT, 1)
    inv_gap = (1.0 / jnp.maximum(jnp.asarray(gap, jnp.float32), 1.0)).reshape(1, 1)

    out, loss = _tc_main(
        obs_rows, gt_rows, opos, npos, inv_gap,
        W_in, W_time, W_q, W_k, W_v, W_o, W_ff1, W_ff2, W_out,
        b_in.reshape(1, _DM), b_ff1.reshape(1, _DI),
        b_ff2.reshape(1, _DM), b_out.reshape(1, _F),
    )
    return (out, loss.reshape(())[()])


# speedup vs baseline: 4.9234x; 1.7761x over previous
import functools

import numpy as np
import jax
import jax.numpy as jnp
from jax import lax
from jax.experimental import pallas as pl
from jax.experimental.pallas import tpu as pltpu
from jax.experimental.pallas import tpu_sc as plsc

_B = 16
_T = 512
_P = 10
_F = 6
_PF = _P * _F
_DM = 64
_TDIM = 64
_NH = 4
_DH = 16
_DI = 128
_NOBS = 256
_NNEXT = 256

_NC = 2
_NS = 16
_NW = _NC * _NS
_ROWS = _B * _NOBS
_RPW = _ROWS // _NW


_SLAB = _T * _PF
_CHUNK = _RPW * _PF
_NGRP = _CHUNK // 16


def _sc_gather(x2d, obs_idx, next_idx):
    mesh = plsc.VectorSubcoreMesh(core_axis_name="c", subcore_axis_name="s")

    @functools.partial(
        pl.kernel,
        mesh=mesh,
        out_type=(
            jax.ShapeDtypeStruct((_ROWS * _PF,), jnp.float32),
            jax.ShapeDtypeStruct((_ROWS * _PF,), jnp.float32),
        ),
        scratch_types=[
            pltpu.VMEM((_SLAB,), jnp.float32),
            pltpu.VMEM((_RPW,), jnp.int32),
            pltpu.VMEM((_RPW,), jnp.int32),
            pltpu.VMEM((_CHUNK,), jnp.float32),
            pltpu.SemaphoreType.DMA,
        ],
        compiler_params=pltpu.CompilerParams(
            use_tc_tiling_on_sc=False, needs_layout_passes=False),
    )
    def k(x_hbm, obs_hbm, nxt_hbm, out_obs, out_gt, slab_v, ti_v, tn_v, buf_v, sem):
        w = lax.axis_index("s") * _NC + lax.axis_index("c")
        b = w // 2
        n0 = pl.multiple_of((w % 2) * _RPW, _RPW)
        base = pl.multiple_of(w * _CHUNK, _CHUNK)
        pltpu.sync_copy(x_hbm.at[b], slab_v)
        pltpu.sync_copy(obs_hbm.at[pl.ds(n0, _RPW)], ti_v)
        pltpu.sync_copy(nxt_hbm.at[pl.ds(n0, _RPW)], tn_v)
        lane = lax.iota(jnp.int32, 16)
        for ti, out_hbm in ((ti_v, out_obs), (tn_v, out_gt)):
            @plsc.parallel_loop(0, _NGRP, step=1, unroll=8)
            def _g(g):
                o = g * 16 + lane
                n = lax.shift_right_logical(o * 34953, 21)
                f = o - n * _PF
                t16 = plsc.load_gather(ti, [n])
                v = plsc.load_gather(slab_v, [t16 * _PF + f])
                buf_v[pl.ds(pl.multiple_of(g * 16, 16), 16)] = v
            pltpu.sync_copy(buf_v, out_hbm.at[pl.ds(base, _CHUNK)])

    return k(x2d.reshape(_B, _SLAB), obs_idx, next_idx)


def _pre_body(opos_r, npos_r, ig_r, w_in_r, w_time_r, w_q_r, w_k_r, w_v_r,
              w_o_r, w_ff1_r, w_ff2_r, w_out_r, b_in_r,
              c_obs_o, qb_o, q_o, wkv_o, w_in_o, w_o_o, w_ff1_o, w_ff2_o,
              w_out_o):
    bf = jnp.bfloat16
    ig = ig_r[0, 0]
    i2 = lax.broadcasted_iota(jnp.int32, (_NOBS, _TDIM // 2), 1).astype(jnp.float32)
    freq = jnp.exp(i2 * jnp.float32(-2.0 * np.log(10000.0) / _TDIM))

    def tenc(pos_col):
        ang = (pos_col * ig) * freq
        return jnp.concatenate([jnp.sin(ang), jnp.cos(ang)], axis=1)

    w_time = w_time_r[...]
    t_obs = tenc(opos_r[...])
    t_next = tenc(npos_r[...])
    c_obs_o[...] = jnp.dot(t_obs, w_time, preferred_element_type=jnp.float32) + b_in_r[...]
    qb = jnp.dot(t_next, w_time, preferred_element_type=jnp.float32)
    qb_o[...] = qb
    qsc = jnp.float32(np.log2(np.e) / np.sqrt(_DH))
    q_o[...] = (jnp.dot(qb, w_q_r[...], preferred_element_type=jnp.float32) * qsc).astype(bf)
    wkv_o[...] = jnp.concatenate([w_k_r[...], w_v_r[...]], axis=1).astype(bf)
    w_in_o[...] = w_in_r[...].astype(bf)
    w_o_o[...] = w_o_r[...].astype(bf)
    w_ff1_o[...] = w_ff1_r[...].astype(bf)
    w_ff2_o[...] = w_ff2_r[...].astype(bf)
    w_out_o[...] = w_out_r[...].astype(bf)


def _precompute(opos, npos, inv_gap, W_in, W_time, W_q, W_k, W_v, W_o,
                W_ff1, W_ff2, W_out, b_in):
    bf = jnp.bfloat16
    out_shape = [
        jax.ShapeDtypeStruct((_NOBS, _DM), jnp.float32),
        jax.ShapeDtypeStruct((_NNEXT, _DM), jnp.float32),
        jax.ShapeDtypeStruct((_NNEXT, _DM), bf),
        jax.ShapeDtypeStruct((_DM, 2 * _DM), bf),
        jax.ShapeDtypeStruct((_F, _DM), bf),
        jax.ShapeDtypeStruct((_DM, _DM), bf),
        jax.ShapeDtypeStruct((_DM, _DI), bf),
        jax.ShapeDtypeStruct((_DI, _DM), bf),
        jax.ShapeDtypeStruct((_DM, _F), bf),
    ]
    return pl.pallas_call(_pre_body, out_shape=out_shape)(
        opos, npos, inv_gap, W_in, W_time, W_q, W_k, W_v, W_o,
        W_ff1, W_ff2, W_out, b_in)


def _tc_body(obs_r, gt_r, c_obs_r, qb_r, q_r, wkv_r,
             w_in_r, w_o_r, w_ff1_r, w_ff2_r, w_out_r,
             b_ff1_r, b_ff2_r, b_out_r,
             out_r, loss_r):
    b = pl.program_id(0)
    bf = jnp.bfloat16

    c_obs = c_obs_r[...]
    q_base = qb_r[...]
    q = q_r[...]
    wkv = wkv_r[...]
    w_in = w_in_r[...]
    w_o = w_o_r[...]
    w_ff1 = w_ff1_r[...]
    w_ff2 = w_ff2_r[...]
    w_out = w_out_r[...]
    b_ff1 = b_ff1_r[...]
    b_ff2 = b_ff2_r[...]
    b_out = b_out_r[...]
    ones_col = jnp.ones((_NOBS, 1), bf)

    def fdot(a, b2):
        return jnp.dot(a, b2, preferred_element_type=jnp.float32)

    obs_all = obs_r[0]
    ops = [obs_all[:, _F * p:_F * (p + 1)].astype(bf) for p in range(_P)]
    h0s = [fdot(op, w_in) + c_obs for op in ops]
    kvs = [fdot(h0.astype(bf), wkv).astype(bf) for h0 in h0s]
    v1s = [jnp.concatenate([kv[:, _DM:], ones_col], axis=1) for kv in kvs]
    qhs = [q[:, _DH * h:_DH * (h + 1)] for h in range(_NH)]
    lgs = [[lax.dot_general(qhs[h], kv[:, _DH * h:_DH * (h + 1)],
                            (((1,), (1,)), ((), ())),
                            preferred_element_type=jnp.float32)
            for h in range(_NH)] for kv in kvs]
    es = [[jnp.exp2(lg.astype(bf)) for lg in row] for row in lgs]
    css = [[fdot(e, v1s[p]) for e in es[p]] for p in range(_P)]
    ctxs = [jnp.concatenate(
        [cs[:, _DH * h:_DH * (h + 1)] * (1.0 / cs[:, _DM:_DM + 1])
         for h, cs in enumerate(css[p])], axis=1) for p in range(_P)]
    h1s = [q_base + fdot(ctx.astype(bf), w_o) for ctx in ctxs]
    ffs = [jnp.maximum(fdot(h1.astype(bf), w_ff1) + b_ff1, 0.0) for h1 in h1s]
    h2s = [h1 + fdot(ff.astype(bf), w_ff2) + b_ff2 for h1, ff in zip(h1s, ffs)]
    outs = [fdot(h2.astype(bf), w_out) + b_out for h2 in h2s]
    o = jnp.concatenate(outs, axis=1)
    out_r[0] = o

    part = jnp.sum(jnp.abs(o - gt_r[0]))

    @pl.when(b == 0)
    def _init():
        loss_r[0, 0] = 0.0

    loss_r[0, 0] += part

    @pl.when(b == _B - 1)
    def _fin():
        loss_r[0, 0] = loss_r[0, 0] * jnp.float32(1.0 / (_B * _P * _NNEXT * _F))


def _tc_main(obs_rows, gt_rows, c_obs, q_base, q16, wkv16,
             w_in16, w_o16, w_ff1_16, w_ff2_16, w_out16,
             b_ff1, b_ff2, b_out):
    bf = jnp.bfloat16
    w2 = lambda b: (0, 0)
    in_specs = [
        pl.BlockSpec((1, _NOBS, _PF), lambda b: (b, 0, 0)),
        pl.BlockSpec((1, _NNEXT, _PF), lambda b: (b, 0, 0)),
        pl.BlockSpec((_NOBS, _DM), w2),
        pl.BlockSpec((_NNEXT, _DM), w2),
        pl.BlockSpec((_NNEXT, _DM), w2),
        pl.BlockSpec((_DM, 2 * _DM), w2),
        pl.BlockSpec((_F, _DM), w2),
        pl.BlockSpec((_DM, _DM), w2),
        pl.BlockSpec((_DM, _DI), w2),
        pl.BlockSpec((_DI, _DM), w2),
        pl.BlockSpec((_DM, _F), w2),
        pl.BlockSpec((1, _DI), w2),
        pl.BlockSpec((1, _DM), w2),
        pl.BlockSpec((1, _F), w2),
    ]
    out_specs = [
        pl.BlockSpec((1, _NNEXT, _PF), lambda b: (b, 0, 0)),
        pl.BlockSpec((1, 1), w2, memory_space=pltpu.SMEM),
    ]
    out_shape = [
        jax.ShapeDtypeStruct((_B, _NNEXT, _PF), jnp.float32),
        jax.ShapeDtypeStruct((1, 1), jnp.float32),
    ]
    return pl.pallas_call(
        _tc_body,
        grid=(_B,),
        in_specs=in_specs,
        out_specs=out_specs,
        out_shape=out_shape,
        compiler_params=pltpu.CompilerParams(
            dimension_semantics=("arbitrary",),
        ),
    )(obs_rows, gt_rows, c_obs, q_base, q16, wkv16,
      w_in16, w_o16, w_ff1_16, w_ff2_16, w_out16,
      b_ff1, b_ff2, b_out)


def kernel(input_data, obs_idx, next_idx, gap,
           W_in, b_in, W_time, W_q, W_k, W_v, W_o,
           W_ff1, b_ff1, W_ff2, b_ff2, W_out, b_out):
    x2d = input_data.reshape(_B * _T, _PF)
    r_obs, r_gt = _sc_gather(x2d, obs_idx, next_idx)
    obs_rows = r_obs.reshape(_B, _NOBS, _PF)
    gt_rows = r_gt.reshape(_B, _NNEXT, _PF)

    opos = obs_idx.astype(jnp.float32).reshape(_NOBS, 1)
    npos = next_idx.astype(jnp.float32).reshape(_NNEXT, 1)
    inv_gap = (1.0 / jnp.maximum(jnp.asarray(gap, jnp.float32), 1.0)).reshape(1, 1)

    pre = _precompute(opos, npos, inv_gap, W_in, W_time, W_q, W_k, W_v, W_o,
                      W_ff1, W_ff2, W_out, b_in.reshape(1, _DM))

    out, loss = _tc_main(
        obs_rows, gt_rows, *pre,
        b_ff1.reshape(1, _DI), b_ff2.reshape(1, _DM), b_out.reshape(1, _F),
    )
    return (out, loss.reshape(())[()])


# speedup vs baseline: 5.0834x; 1.0325x over previous
import functools

import numpy as np
import jax
import jax.numpy as jnp
from jax import lax
from jax.experimental import pallas as pl
from jax.experimental.pallas import tpu as pltpu
from jax.experimental.pallas import tpu_sc as plsc

_B = 16
_T = 512
_P = 10
_F = 6
_PF = _P * _F
_DM = 64
_TDIM = 64
_NH = 4
_DH = 16
_DI = 128
_NOBS = 256
_NNEXT = 256

_NC = 2
_NS = 16
_NW = _NC * _NS
_ROWS = _B * _NOBS
_RPW = _ROWS // _NW


_SLAB = _T * _PF
_CHUNK = _RPW * _PF
_NGRP = _CHUNK // 16


def _sc_gather(x2d, obs_idx, next_idx):
    mesh = plsc.VectorSubcoreMesh(core_axis_name="c", subcore_axis_name="s")

    @functools.partial(
        pl.kernel,
        mesh=mesh,
        out_type=(
            jax.ShapeDtypeStruct((_ROWS * _PF,), jnp.float32),
            jax.ShapeDtypeStruct((_ROWS * _PF,), jnp.float32),
        ),
        scratch_types=[
            pltpu.VMEM((_SLAB,), jnp.float32),
            pltpu.VMEM((_RPW,), jnp.int32),
            pltpu.VMEM((_RPW,), jnp.int32),
            pltpu.VMEM((_CHUNK,), jnp.float32),
            pltpu.SemaphoreType.DMA,
        ],
        compiler_params=pltpu.CompilerParams(
            use_tc_tiling_on_sc=False, needs_layout_passes=False),
    )
    def k(x_hbm, obs_hbm, nxt_hbm, out_obs, out_gt, slab_v, ti_v, tn_v, buf_v, sem):
        w = lax.axis_index("s") * _NC + lax.axis_index("c")
        b = w // 2
        n0 = pl.multiple_of((w % 2) * _RPW, _RPW)
        base = pl.multiple_of(w * _CHUNK, _CHUNK)
        pltpu.sync_copy(x_hbm.at[b], slab_v)
        pltpu.sync_copy(obs_hbm.at[pl.ds(n0, _RPW)], ti_v)
        pltpu.sync_copy(nxt_hbm.at[pl.ds(n0, _RPW)], tn_v)
        lane = lax.iota(jnp.int32, 16)
        for ti, out_hbm in ((ti_v, out_obs), (tn_v, out_gt)):
            @plsc.parallel_loop(0, _NGRP, step=1, unroll=8)
            def _g(g):
                o = g * 16 + lane
                n = lax.shift_right_logical(o * 34953, 21)
                f = o - n * _PF
                t16 = plsc.load_gather(ti, [n])
                v = plsc.load_gather(slab_v, [t16 * _PF + f])
                buf_v[pl.ds(pl.multiple_of(g * 16, 16), 16)] = v
            pltpu.sync_copy(buf_v, out_hbm.at[pl.ds(base, _CHUNK)])

    return k(x2d.reshape(_B, _SLAB), obs_idx, next_idx)


def _pre_body(oidx_r, nidx_r, gap_r, w_in_r, w_time_r, w_q_r, w_k_r, w_v_r,
              w_o_r, w_ff1_r, w_ff2_r, w_out_r, b_in_r, b_out_r,
              c_obs10_o, qb_o, q_o, wkv_o, w_in_bd_o, w_o_o, w_ff1_o,
              w_ff2_o, w_out_bd_o, b_out60_o):
    bf = jnp.bfloat16
    ig = 1.0 / jnp.maximum(gap_r[0, 0].astype(jnp.float32), 1.0)
    i2 = lax.broadcasted_iota(jnp.int32, (_NOBS, _TDIM // 2), 1).astype(jnp.float32)
    freq = jnp.exp(i2 * jnp.float32(-2.0 * np.log(10000.0) / _TDIM))

    def tenc(pos_col):
        ang = (pos_col * ig) * freq
        return jnp.concatenate([jnp.sin(ang), jnp.cos(ang)], axis=1)

    w_time = w_time_r[...]
    t_obs = tenc(oidx_r[...].astype(jnp.float32))
    t_next = tenc(nidx_r[...].astype(jnp.float32))
    c_obs = jnp.dot(t_obs, w_time, preferred_element_type=jnp.float32) + b_in_r[...]
    c_obs10_o[...] = jnp.concatenate([c_obs] * _P, axis=1)
    qb = jnp.dot(t_next, w_time, preferred_element_type=jnp.float32)
    qb_o[...] = qb
    qsc = jnp.float32(np.log2(np.e) / np.sqrt(_DH))
    q_o[...] = (jnp.dot(qb, w_q_r[...], preferred_element_type=jnp.float32) * qsc).astype(bf)
    wkv_o[...] = jnp.concatenate([w_k_r[...], w_v_r[...]], axis=1).astype(bf)
    w_in16 = w_in_r[...].astype(bf)
    zin = jnp.zeros((_F, _DM), bf)
    rows = []
    for p in range(_P):
        pieces = [w_in16 if j == p else zin for j in range(_P)]
        rows.append(jnp.concatenate(pieces, axis=1))
    w_in_bd_o[...] = jnp.concatenate(rows, axis=0)
    w_out16 = w_out_r[...].astype(bf)
    zout = jnp.zeros((_DM, _F), bf)
    orows = []
    for p in range(_P):
        pieces = [w_out16 if j == p else zout for j in range(_P)]
        orows.append(jnp.concatenate(pieces, axis=1))
    w_out_bd_o[...] = jnp.concatenate(orows, axis=0)
    b_out60_o[...] = jnp.concatenate([b_out_r[...]] * _P, axis=1)
    w_o_o[...] = w_o_r[...].astype(bf)
    w_ff1_o[...] = w_ff1_r[...].astype(bf)
    w_ff2_o[...] = w_ff2_r[...].astype(bf)


def _precompute(oidx, nidx, gap2, W_in, W_time, W_q, W_k, W_v, W_o,
                W_ff1, W_ff2, W_out, b_in, b_out):
    bf = jnp.bfloat16
    out_shape = [
        jax.ShapeDtypeStruct((_NOBS, _P * _DM), jnp.float32),
        jax.ShapeDtypeStruct((_NNEXT, _DM), jnp.float32),
        jax.ShapeDtypeStruct((_NNEXT, _DM), bf),
        jax.ShapeDtypeStruct((_DM, 2 * _DM), bf),
        jax.ShapeDtypeStruct((_PF, _P * _DM), bf),
        jax.ShapeDtypeStruct((_DM, _DM), bf),
        jax.ShapeDtypeStruct((_DM, _DI), bf),
        jax.ShapeDtypeStruct((_DI, _DM), bf),
        jax.ShapeDtypeStruct((_P * _DM, _PF), bf),
        jax.ShapeDtypeStruct((1, _PF), jnp.float32),
    ]
    return pl.pallas_call(_pre_body, out_shape=out_shape)(
        oidx, nidx, gap2, W_in, W_time, W_q, W_k, W_v, W_o,
        W_ff1, W_ff2, W_out, b_in, b_out)


def _tc_body(obs_r, gt_r, c_obs10_r, qb_r, q_r, wkv_r,
             w_in_bd_r, w_o_r, w_ff1_r, w_ff2_r, w_out_bd_r,
             b_ff1_r, b_ff2_r, b_out60_r,
             out_r, loss_r):
    b = pl.program_id(0)
    bf = jnp.bfloat16

    c_obs10 = c_obs10_r[...]
    q_base = qb_r[...]
    q = q_r[...]
    wkv = wkv_r[...]
    w_in_bd = w_in_bd_r[...]
    w_o = w_o_r[...]
    w_ff1 = w_ff1_r[...]
    w_ff2 = w_ff2_r[...]
    w_out_bd = w_out_bd_r[...]
    b_ff1 = b_ff1_r[...]
    b_ff2 = b_ff2_r[...]
    b_out60 = b_out60_r[...]
    ones_col = jnp.ones((_NOBS, 1), bf)

    def fdot(a, b2):
        return jnp.dot(a, b2, preferred_element_type=jnp.float32)

    obs_all = obs_r[0]
    h0_all = (fdot(obs_all.astype(bf), w_in_bd) + c_obs10).astype(bf)
    h0s = [h0_all[:, _DM * p:_DM * (p + 1)] for p in range(_P)]
    kvs = [fdot(h0, wkv).astype(bf) for h0 in h0s]
    v1s = [jnp.concatenate([kv[:, _DM:], ones_col], axis=1) for kv in kvs]
    qhs = [q[:, _DH * h:_DH * (h + 1)] for h in range(_NH)]
    kstk = [jnp.concatenate([kv[:, _DH * h:_DH * (h + 1)] for kv in kvs],
                            axis=0) for h in range(_NH)]
    lgs = [lax.dot_general(qhs[h], kstk[h], (((1,), (1,)), ((), ())),
                           preferred_element_type=jnp.float32)
           for h in range(_NH)]
    es = [jnp.exp2(lg.astype(bf)) for lg in lgs]
    css = [[fdot(es[h][:, _NOBS * p:_NOBS * (p + 1)], v1s[p])
            for h in range(_NH)] for p in range(_P)]
    ctxs = [jnp.concatenate(
        [cs[:, _DH * h:_DH * (h + 1)] * (1.0 / cs[:, _DM:_DM + 1])
         for h, cs in enumerate(css[p])], axis=1) for p in range(_P)]
    h1s = [q_base + fdot(ctx.astype(bf), w_o) for ctx in ctxs]
    ffs = [jnp.maximum(fdot(h1.astype(bf), w_ff1) + b_ff1, 0.0) for h1 in h1s]
    h2s = [h1 + fdot(ff.astype(bf), w_ff2) + b_ff2 for h1, ff in zip(h1s, ffs)]
    h2_all = jnp.concatenate([h2.astype(bf) for h2 in h2s], axis=1)
    o = fdot(h2_all, w_out_bd) + b_out60
    out_r[0] = o

    part = jnp.sum(jnp.abs(o - gt_r[0]))

    @pl.when(b == 0)
    def _init():
        loss_r[0, 0] = 0.0

    loss_r[0, 0] += part

    @pl.when(b == _B - 1)
    def _fin():
        loss_r[0, 0] = loss_r[0, 0] * jnp.float32(1.0 / (_B * _P * _NNEXT * _F))


def _tc_main(obs_rows, gt_rows, c_obs10, q_base, q16, wkv16,
             w_in_bd, w_o16, w_ff1_16, w_ff2_16, w_out_bd,
             b_ff1, b_ff2, b_out60):
    bf = jnp.bfloat16
    w2 = lambda b: (0, 0)
    in_specs = [
        pl.BlockSpec((1, _NOBS, _PF), lambda b: (b, 0, 0)),
        pl.BlockSpec((1, _NNEXT, _PF), lambda b: (b, 0, 0)),
        pl.BlockSpec((_NOBS, _P * _DM), w2),
        pl.BlockSpec((_NNEXT, _DM), w2),
        pl.BlockSpec((_NNEXT, _DM), w2),
        pl.BlockSpec((_DM, 2 * _DM), w2),
        pl.BlockSpec((_PF, _P * _DM), w2),
        pl.BlockSpec((_DM, _DM), w2),
        pl.BlockSpec((_DM, _DI), w2),
        pl.BlockSpec((_DI, _DM), w2),
        pl.BlockSpec((_P * _DM, _PF), w2),
        pl.BlockSpec((1, _DI), w2),
        pl.BlockSpec((1, _DM), w2),
        pl.BlockSpec((1, _PF), w2),
    ]
    out_specs = [
        pl.BlockSpec((1, _NNEXT, _PF), lambda b: (b, 0, 0)),
        pl.BlockSpec((1, 1), w2, memory_space=pltpu.SMEM),
    ]
    out_shape = [
        jax.ShapeDtypeStruct((_B, _NNEXT, _PF), jnp.float32),
        jax.ShapeDtypeStruct((1, 1), jnp.float32),
    ]
    return pl.pallas_call(
        _tc_body,
        grid=(_B,),
        in_specs=in_specs,
        out_specs=out_specs,
        out_shape=out_shape,
        compiler_params=pltpu.CompilerParams(
            dimension_semantics=("arbitrary",),
        ),
    )(obs_rows, gt_rows, c_obs10, q_base, q16, wkv16,
      w_in_bd, w_o16, w_ff1_16, w_ff2_16, w_out_bd,
      b_ff1, b_ff2, b_out60)


def kernel(input_data, obs_idx, next_idx, gap,
           W_in, b_in, W_time, W_q, W_k, W_v, W_o,
           W_ff1, b_ff1, W_ff2, b_ff2, W_out, b_out):
    x2d = input_data.reshape(_B * _T, _PF)
    r_obs, r_gt = _sc_gather(x2d, obs_idx, next_idx)
    obs_rows = r_obs.reshape(_B, _NOBS, _PF)
    gt_rows = r_gt.reshape(_B, _NNEXT, _PF)

    gap2 = jnp.asarray(gap, jnp.int32).reshape(1, 1)
    pre = _precompute(obs_idx.reshape(_NOBS, 1), next_idx.reshape(_NNEXT, 1),
                      gap2, W_in, W_time, W_q, W_k, W_v, W_o,
                      W_ff1, W_ff2, W_out, b_in.reshape(1, _DM),
                      b_out.reshape(1, _F))

    out, loss = _tc_main(
        obs_rows, gt_rows, *pre[:9],
        b_ff1.reshape(1, _DI), b_ff2.reshape(1, _DM), pre[9],
    )
    return (out, loss.reshape(())[()])


# speedup vs baseline: 5.2197x; 1.0268x over previous
import functools

import numpy as np
import jax
import jax.numpy as jnp
from jax import lax
from jax.experimental import pallas as pl
from jax.experimental.pallas import tpu as pltpu
from jax.experimental.pallas import tpu_sc as plsc

_B = 16
_T = 512
_P = 10
_F = 6
_PF = _P * _F
_DM = 64
_TDIM = 64
_NH = 4
_DH = 16
_DI = 128
_NOBS = 256
_NNEXT = 256

_BPG = 4

_NC = 2
_NS = 16
_NW = _NC * _NS
_ROWS = _B * _NOBS
_RPW = _ROWS // _NW


_SLAB = _T * _PF
_CHUNK = _RPW * _PF
_NGRP = _CHUNK // 16


def _sc_gather(x2d, obs_idx, next_idx):
    mesh = plsc.VectorSubcoreMesh(core_axis_name="c", subcore_axis_name="s")

    @functools.partial(
        pl.kernel,
        mesh=mesh,
        out_type=(
            jax.ShapeDtypeStruct((_ROWS * _PF,), jnp.float32),
            jax.ShapeDtypeStruct((_ROWS * _PF,), jnp.float32),
        ),
        scratch_types=[
            pltpu.VMEM((_SLAB,), jnp.float32),
            pltpu.VMEM((_RPW,), jnp.int32),
            pltpu.VMEM((_RPW,), jnp.int32),
            pltpu.VMEM((_CHUNK,), jnp.float32),
            pltpu.SemaphoreType.DMA,
        ],
        compiler_params=pltpu.CompilerParams(
            use_tc_tiling_on_sc=False, needs_layout_passes=False),
    )
    def k(x_hbm, obs_hbm, nxt_hbm, out_obs, out_gt, slab_v, ti_v, tn_v, buf_v, sem):
        w = lax.axis_index("s") * _NC + lax.axis_index("c")
        b = w // 2
        n0 = pl.multiple_of((w % 2) * _RPW, _RPW)
        base = pl.multiple_of(w * _CHUNK, _CHUNK)
        pltpu.sync_copy(x_hbm.at[b], slab_v)
        pltpu.sync_copy(obs_hbm.at[pl.ds(n0, _RPW)], ti_v)
        pltpu.sync_copy(nxt_hbm.at[pl.ds(n0, _RPW)], tn_v)
        lane = lax.iota(jnp.int32, 16)
        for ti, out_hbm in ((ti_v, out_obs), (tn_v, out_gt)):
            @plsc.parallel_loop(0, _NGRP, step=1, unroll=8)
            def _g(g):
                o = g * 16 + lane
                n = lax.shift_right_logical(o * 34953, 21)
                f = o - n * _PF
                t16 = plsc.load_gather(ti, [n])
                v = plsc.load_gather(slab_v, [t16 * _PF + f])
                buf_v[pl.ds(pl.multiple_of(g * 16, 16), 16)] = v
            pltpu.sync_copy(buf_v, out_hbm.at[pl.ds(base, _CHUNK)])

    return k(x2d.reshape(_B, _SLAB), obs_idx, next_idx)


def _pre_body(oidx_r, nidx_r, gap_r, w_in_r, w_time_r, w_q_r, w_k_r, w_v_r,
              w_o_r, w_ff1_r, w_ff2_r, w_out_r, b_in_r, b_out_r,
              c_obs10_o, qb_o, q_o, wkv_o, w_in_bd_o, w_o_o, w_ff1_o,
              w_ff2_o, w_out_bd_o, b_out60_o):
    bf = jnp.bfloat16
    ig = 1.0 / jnp.maximum(gap_r[0, 0].astype(jnp.float32), 1.0)
    i2 = lax.broadcasted_iota(jnp.int32, (_NOBS, _TDIM // 2), 1).astype(jnp.float32)
    freq = jnp.exp(i2 * jnp.float32(-2.0 * np.log(10000.0) / _TDIM))

    def tenc(pos_col):
        ang = (pos_col * ig) * freq
        return jnp.concatenate([jnp.sin(ang), jnp.cos(ang)], axis=1)

    w_time = w_time_r[...]
    t_obs = tenc(oidx_r[...].astype(jnp.float32))
    t_next = tenc(nidx_r[...].astype(jnp.float32))
    c_obs = jnp.dot(t_obs, w_time, preferred_element_type=jnp.float32) + b_in_r[...]
    c_obs10_o[...] = jnp.concatenate([c_obs] * _P, axis=1)
    qb = jnp.dot(t_next, w_time, preferred_element_type=jnp.float32)
    qb_o[...] = qb
    qsc = jnp.float32(np.log2(np.e) / np.sqrt(_DH))
    q_o[...] = (jnp.dot(qb, w_q_r[...], preferred_element_type=jnp.float32) * qsc).astype(bf)
    wkv_o[...] = jnp.concatenate([w_k_r[...], w_v_r[...]], axis=1).astype(bf)
    w_in16 = w_in_r[...].astype(bf)
    zin = jnp.zeros((_F, _DM), bf)
    rows = []
    for p in range(_P):
        pieces = [w_in16 if j == p else zin for j in range(_P)]
        rows.append(jnp.concatenate(pieces, axis=1))
    w_in_bd_o[...] = jnp.concatenate(rows, axis=0)
    w_out16 = w_out_r[...].astype(bf)
    zout = jnp.zeros((_DM, _F), bf)
    orows = []
    for p in range(_P):
        pieces = [w_out16 if j == p else zout for j in range(_P)]
        orows.append(jnp.concatenate(pieces, axis=1))
    w_out_bd_o[...] = jnp.concatenate(orows, axis=0)
    b_out60_o[...] = jnp.concatenate([b_out_r[...]] * _P, axis=1)
    w_o_o[...] = w_o_r[...].astype(bf)
    w_ff1_o[...] = w_ff1_r[...].astype(bf)
    w_ff2_o[...] = w_ff2_r[...].astype(bf)


def _precompute(oidx, nidx, gap2, W_in, W_time, W_q, W_k, W_v, W_o,
                W_ff1, W_ff2, W_out, b_in, b_out):
    bf = jnp.bfloat16
    out_shape = [
        jax.ShapeDtypeStruct((_NOBS, _P * _DM), jnp.float32),
        jax.ShapeDtypeStruct((_NNEXT, _DM), jnp.float32),
        jax.ShapeDtypeStruct((_NNEXT, _DM), bf),
        jax.ShapeDtypeStruct((_DM, 2 * _DM), bf),
        jax.ShapeDtypeStruct((_PF, _P * _DM), bf),
        jax.ShapeDtypeStruct((_DM, _DM), bf),
        jax.ShapeDtypeStruct((_DM, _DI), bf),
        jax.ShapeDtypeStruct((_DI, _DM), bf),
        jax.ShapeDtypeStruct((_P * _DM, _PF), bf),
        jax.ShapeDtypeStruct((1, _PF), jnp.float32),
    ]
    return pl.pallas_call(_pre_body, out_shape=out_shape)(
        oidx, nidx, gap2, W_in, W_time, W_q, W_k, W_v, W_o,
        W_ff1, W_ff2, W_out, b_in, b_out)


def _tc_body(obs_r, gt_r, c_obs10_r, qb_r, q_r, wkv_r,
             w_in_bd_r, w_o_r, w_ff1_r, w_ff2_r, w_out_bd_r,
             b_ff1_r, b_ff2_r, b_out60_r,
             out_r, loss_r):
    b = pl.program_id(0)
    bf = jnp.bfloat16

    c_obs10 = c_obs10_r[...]
    q_base = qb_r[...]
    q = q_r[...]
    wkv = wkv_r[...]
    w_in_bd = w_in_bd_r[...]
    w_o = w_o_r[...]
    w_ff1 = w_ff1_r[...]
    w_ff2 = w_ff2_r[...]
    w_out_bd = w_out_bd_r[...]
    b_ff1 = b_ff1_r[...]
    b_ff2 = b_ff2_r[...]
    b_out60 = b_out60_r[...]
    ones_col = jnp.ones((_NOBS, 1), bf)

    def fdot(a, b2):
        return jnp.dot(a, b2, preferred_element_type=jnp.float32)

    part = jnp.float32(0.0)
    for bi in range(_BPG):
        part = part + _one_batch(obs_r, gt_r, out_r, bi, c_obs10, q_base, q,
                                 wkv, w_in_bd, w_o, w_ff1, w_ff2, w_out_bd,
                                 b_ff1, b_ff2, b_out60, ones_col, fdot, bf)

    @pl.when(b == 0)
    def _init():
        loss_r[0, 0] = 0.0

    loss_r[0, 0] += part

    @pl.when(b == _B // _BPG - 1)
    def _fin():
        loss_r[0, 0] = loss_r[0, 0] * jnp.float32(1.0 / (_B * _P * _NNEXT * _F))


def _one_batch(obs_r, gt_r, out_r, bi, c_obs10, q_base, q, wkv, w_in_bd,
               w_o, w_ff1, w_ff2, w_out_bd, b_ff1, b_ff2, b_out60,
               ones_col, fdot, bf):
    obs_all = obs_r[bi]
    h0_all = (fdot(obs_all.astype(bf), w_in_bd) + c_obs10).astype(bf)
    h0s = [h0_all[:, _DM * p:_DM * (p + 1)] for p in range(_P)]
    kvs = [fdot(h0, wkv).astype(bf) for h0 in h0s]
    v1s = [jnp.concatenate([kv[:, _DM:], ones_col], axis=1) for kv in kvs]
    qhs = [q[:, _DH * h:_DH * (h + 1)] for h in range(_NH)]
    kstk = [jnp.concatenate([kv[:, _DH * h:_DH * (h + 1)] for kv in kvs],
                            axis=0) for h in range(_NH)]
    lgs = [lax.dot_general(qhs[h], kstk[h], (((1,), (1,)), ((), ())),
                           preferred_element_type=jnp.float32)
           for h in range(_NH)]
    es = [jnp.exp2(lg.astype(bf)) for lg in lgs]
    css = [[fdot(es[h][:, _NOBS * p:_NOBS * (p + 1)], v1s[p])
            for h in range(_NH)] for p in range(_P)]
    ctxs = [jnp.concatenate(
        [cs[:, _DH * h:_DH * (h + 1)] * (1.0 / cs[:, _DM:_DM + 1])
         for h, cs in enumerate(css[p])], axis=1) for p in range(_P)]
    h1s = [q_base + fdot(ctx.astype(bf), w_o) for ctx in ctxs]
    ffs = [jnp.maximum(fdot(h1.astype(bf), w_ff1) + b_ff1, 0.0) for h1 in h1s]
    h2s = [h1 + fdot(ff.astype(bf), w_ff2) + b_ff2 for h1, ff in zip(h1s, ffs)]
    h2_all = jnp.concatenate([h2.astype(bf) for h2 in h2s], axis=1)
    o = fdot(h2_all, w_out_bd) + b_out60
    out_r[bi] = o
    return jnp.sum(jnp.abs(o - gt_r[bi]))


def _tc_main(obs_rows, gt_rows, c_obs10, q_base, q16, wkv16,
             w_in_bd, w_o16, w_ff1_16, w_ff2_16, w_out_bd,
             b_ff1, b_ff2, b_out60):
    bf = jnp.bfloat16
    w2 = lambda b: (0, 0)
    in_specs = [
        pl.BlockSpec((_BPG, _NOBS, _PF), lambda b: (b, 0, 0)),
        pl.BlockSpec((_BPG, _NNEXT, _PF), lambda b: (b, 0, 0)),
        pl.BlockSpec((_NOBS, _P * _DM), w2),
        pl.BlockSpec((_NNEXT, _DM), w2),
        pl.BlockSpec((_NNEXT, _DM), w2),
        pl.BlockSpec((_DM, 2 * _DM), w2),
        pl.BlockSpec((_PF, _P * _DM), w2),
        pl.BlockSpec((_DM, _DM), w2),
        pl.BlockSpec((_DM, _DI), w2),
        pl.BlockSpec((_DI, _DM), w2),
        pl.BlockSpec((_P * _DM, _PF), w2),
        pl.BlockSpec((1, _DI), w2),
        pl.BlockSpec((1, _DM), w2),
        pl.BlockSpec((1, _PF), w2),
    ]
    out_specs = [
        pl.BlockSpec((_BPG, _NNEXT, _PF), lambda b: (b, 0, 0)),
        pl.BlockSpec((1, 1), w2, memory_space=pltpu.SMEM),
    ]
    out_shape = [
        jax.ShapeDtypeStruct((_B, _NNEXT, _PF), jnp.float32),
        jax.ShapeDtypeStruct((1, 1), jnp.float32),
    ]
    return pl.pallas_call(
        _tc_body,
        grid=(_B // _BPG,),
        in_specs=in_specs,
        out_specs=out_specs,
        out_shape=out_shape,
        compiler_params=pltpu.CompilerParams(
            dimension_semantics=("arbitrary",),
        ),
    )(obs_rows, gt_rows, c_obs10, q_base, q16, wkv16,
      w_in_bd, w_o16, w_ff1_16, w_ff2_16, w_out_bd,
      b_ff1, b_ff2, b_out60)


def kernel(input_data, obs_idx, next_idx, gap,
           W_in, b_in, W_time, W_q, W_k, W_v, W_o,
           W_ff1, b_ff1, W_ff2, b_ff2, W_out, b_out):
    x2d = input_data.reshape(_B * _T, _PF)
    r_obs, r_gt = _sc_gather(x2d, obs_idx, next_idx)
    obs_rows = r_obs.reshape(_B, _NOBS, _PF)
    gt_rows = r_gt.reshape(_B, _NNEXT, _PF)

    gap2 = jnp.asarray(gap, jnp.int32).reshape(1, 1)
    pre = _precompute(obs_idx.reshape(_NOBS, 1), next_idx.reshape(_NNEXT, 1),
                      gap2, W_in, W_time, W_q, W_k, W_v, W_o,
                      W_ff1, W_ff2, W_out, b_in.reshape(1, _DM),
                      b_out.reshape(1, _F))

    out, loss = _tc_main(
        obs_rows, gt_rows, *pre[:9],
        b_ff1.reshape(1, _DI), b_ff2.reshape(1, _DM), pre[9],
    )
    return (out, loss.reshape(())[()])
